# baseline (speedup 1.0000x reference)
import functools

import jax
import jax.numpy as jnp
from jax import lax
from jax.experimental import pallas as pl
from jax.experimental.pallas import tpu as pltpu
from jax.experimental.pallas import tpu_sc as plsc


def _prep_body(h_ref, norm_ref, hn_ref):
    h = h_ref[...]
    sq = jnp.sum(h * h, axis=-1, keepdims=True)
    hn_ref[...] = h * lax.rsqrt(jnp.maximum(sq, 1e-12)) * norm_ref[...]


def _prep(h, norm, block_rows):
    n, d = h.shape
    grid = n // block_rows
    return pl.pallas_call(
        _prep_body,
        grid=(grid,),
        in_specs=[
            pl.BlockSpec((block_rows, d), lambda i: (i, 0)),
            pl.BlockSpec((block_rows, 1), lambda i: (i, 0)),
        ],
        out_specs=pl.BlockSpec((block_rows, d), lambda i: (i, 0)),
        out_shape=jax.ShapeDtypeStruct((n, d), jnp.float32),
    )(h, norm)


_TROWS = 320
_CHUNK = 1600
_GB = 16
_SROWS = 2560
_CAP = 12288
_SEG = 1600
_VL = 16


def _sc_body(n_pad, d, e, hn, srce, dste, agg_out, mx_out, cnt_out,
             sbufa, dbufa, sbufb, dbufb, srcq, rowq, pkg, rows,
             agg_acc, mx_acc, cnt_acc, hn_sh, sem, csem):
    nc = 2
    sid = lax.axis_index("s")
    wid = sid * nc + lax.axis_index("c")
    base = wid * _TROWS
    dsub = d // _VL
    nch = e // _CHUNK
    npair = nch // 2
    nrounds = n_pad // _SROWS

    zf = jnp.zeros((_VL,), jnp.float32)
    ninf = jnp.full((_VL,), -2.0, jnp.float32)

    def init_row(i, _):
        for k in range(dsub):
            agg_acc[i, pl.ds(k * _VL, _VL)] = zf
            mx_acc[i, pl.ds(k * _VL, _VL)] = ninf
        return 0

    lax.fori_loop(0, _TROWS, init_row, 0)

    def init_cnt(i, _):
        cnt_acc[pl.ds(i * _VL, _VL)] = zf
        return 0

    lax.fori_loop(0, (_TROWS + _VL) // _VL, init_cnt, 0)

    one_hot = (lax.iota(jnp.int32, _VL) == 0).astype(jnp.float32)

    def run_batches(qf):
        srcq[pl.ds(qf, _VL)] = jnp.zeros((_VL,), jnp.int32)

        def batch_body(b, _):
            g = b * _GB
            pltpu.async_copy(hn_sh.at[srcq.at[pl.ds(g, _GB)]], rows, sem).wait()
            lim = jnp.minimum(qf - g, _GB)

            def edge_body(j, _):
                rr = rowq[pl.ds(g + j, _VL)][0]
                for k in range(dsub):
                    sl = pl.ds(k * _VL, _VL)
                    m = rows[j, sl]
                    plsc.addupdate(agg_acc.at[rr, sl], m)
                    mx_acc[rr, sl] = jnp.maximum(mx_acc[rr, sl], m)
                plsc.addupdate(cnt_acc.at[pl.ds(rr, _VL)], one_hot)
                return 0

            lax.fori_loop(0, lim, edge_body, 0)
            return 0

        lax.fori_loop(0, (qf + _GB - 1) // _GB, batch_body, 0)

    def stage_block(rbase):
        plsc.subcore_barrier()

        @pl.when(sid == 0)
        def _stage():
            pltpu.sync_copy(hn.at[pl.ds(rbase, _SROWS)], hn_sh)

        plsc.subcore_barrier()

    def filt0_chunk(sb, db, qn):
        def filt(i, qn):
            for h in range(2):
                sl = pl.ds((2 * i + h) * _VL, _VL)
                s = sb[sl]
                delta = db[sl] - base
                mask = plsc.bitcast(delta, jnp.uint32) < jnp.uint32(_TROWS)
                off = jnp.minimum(qn, _CAP)
                pk = s * 512 + delta
                plsc.store_compressed(pkg.at[pl.ds(off, _VL)], pk, mask=mask)
                qn = qn + plsc.all_reduce_population_count(mask)[0]
            return qn

        return lax.fori_loop(0, _CHUNK // (2 * _VL), filt, qn)

    pltpu.async_copy(srce.at[pl.ds(0, _CHUNK)], sbufa, csem)
    pltpu.async_copy(dste.at[pl.ds(0, _CHUNK)], dbufa, csem)

    def pair0_body(i, qn):
        off = i * 2 * _CHUNK
        pltpu.make_async_copy(srce.at[pl.ds(off, _CHUNK)], sbufa, csem).wait()
        pltpu.make_async_copy(dste.at[pl.ds(off, _CHUNK)], dbufa, csem).wait()
        pltpu.async_copy(srce.at[pl.ds(off + _CHUNK, _CHUNK)], sbufb, csem)
        pltpu.async_copy(dste.at[pl.ds(off + _CHUNK, _CHUNK)], dbufb, csem)
        qn = filt0_chunk(sbufa, dbufa, qn)
        pltpu.make_async_copy(srce.at[pl.ds(off + _CHUNK, _CHUNK)], sbufb, csem).wait()
        pltpu.make_async_copy(dste.at[pl.ds(off + _CHUNK, _CHUNK)], dbufb, csem).wait()

        @pl.when(i + 1 < npair)
        def _prefetch():
            noff = off + 2 * _CHUNK
            pltpu.async_copy(srce.at[pl.ds(noff, _CHUNK)], sbufa, csem)
            pltpu.async_copy(dste.at[pl.ds(noff, _CHUNK)], dbufa, csem)

        qn = filt0_chunk(sbufb, dbufb, qn)
        return qn

    qn_total = lax.fori_loop(0, npair, pair0_body, 0)
    pkg[pl.ds(jnp.minimum(qn_total, _CAP), _VL)] = jnp.full((_VL,), -1, jnp.int32)

    def fast_path():
        def round_body(rnd, _):
            rbase = rnd * _SROWS
            stage_block(rbase)
            nseg = (qn_total + _SEG - 1) // _SEG

            def seg_body(sg, _):
                soff = sg * _SEG
                rem = jnp.minimum(qn_total - soff, _SEG)
                nit = (rem + _VL - 1) // _VL

                def filt(i, qf):
                    pk = pkg[pl.ds(soff + i * _VL, _VL)]
                    s = lax.shift_right_arithmetic(pk, 9) - rbase
                    rw = pk & 511
                    mask = plsc.bitcast(s, jnp.uint32) < jnp.uint32(_SROWS)
                    plsc.store_compressed(srcq.at[pl.ds(qf, _VL)], s, mask=mask)
                    plsc.store_compressed(rowq.at[pl.ds(qf, _VL)], rw, mask=mask)
                    return qf + plsc.all_reduce_population_count(mask)[0]

                qf = lax.fori_loop(0, nit, filt, 0)
                run_batches(qf)
                return 0

            lax.fori_loop(0, nseg, seg_body, 0)
            return 0

        lax.fori_loop(0, nrounds, round_body, 0)

    def slow_path():
        def process_chunk(sb, db, rbase):
            def filt(i, qf):
                s = sb[pl.ds(i * _VL, _VL)] - rbase
                dv = db[pl.ds(i * _VL, _VL)]
                mask = ((dv >= base) & (dv < base + _TROWS)
                        & (s >= 0) & (s < _SROWS))
                plsc.store_compressed(srcq.at[pl.ds(qf, _VL)], s, mask=mask)
                plsc.store_compressed(rowq.at[pl.ds(qf, _VL)], dv - base, mask=mask)
                return qf + plsc.all_reduce_population_count(mask)[0]

            qf = lax.fori_loop(0, _CHUNK // _VL, filt, 0)
            run_batches(qf)

        def round_body(rnd, _):
            rbase = rnd * _SROWS
            stage_block(rbase)
            pltpu.async_copy(srce.at[pl.ds(0, _CHUNK)], sbufa, csem)
            pltpu.async_copy(dste.at[pl.ds(0, _CHUNK)], dbufa, csem)

            def pair_body(i, _):
                off = i * 2 * _CHUNK
                pltpu.make_async_copy(srce.at[pl.ds(off, _CHUNK)], sbufa, csem).wait()
                pltpu.make_async_copy(dste.at[pl.ds(off, _CHUNK)], dbufa, csem).wait()
                pltpu.async_copy(srce.at[pl.ds(off + _CHUNK, _CHUNK)], sbufb, csem)
                pltpu.async_copy(dste.at[pl.ds(off + _CHUNK, _CHUNK)], dbufb, csem)
                process_chunk(sbufa, dbufa, rbase)
                pltpu.make_async_copy(srce.at[pl.ds(off + _CHUNK, _CHUNK)], sbufb, csem).wait()
                pltpu.make_async_copy(dste.at[pl.ds(off + _CHUNK, _CHUNK)], dbufb, csem).wait()

                @pl.when(i + 1 < npair)
                def _prefetch():
                    noff = off + 2 * _CHUNK
                    pltpu.async_copy(srce.at[pl.ds(noff, _CHUNK)], sbufa, csem)
                    pltpu.async_copy(dste.at[pl.ds(noff, _CHUNK)], dbufa, csem)

                process_chunk(sbufb, dbufb, rbase)
                return 0

            lax.fori_loop(0, npair, pair_body, 0)
            return 0

        lax.fori_loop(0, nrounds, round_body, 0)

    lax.cond(qn_total > _CAP, slow_path, fast_path)

    pltpu.sync_copy(agg_acc.at[pl.ds(0, _TROWS)], agg_out.at[pl.ds(base, _TROWS)])
    pltpu.sync_copy(mx_acc.at[pl.ds(0, _TROWS)], mx_out.at[pl.ds(base, _TROWS)])
    pltpu.sync_copy(cnt_acc.at[pl.ds(0, _TROWS)], cnt_out.at[pl.ds(base, _TROWS)])


def _sc_segment(hn, src, dst):
    n, d = hn.shape
    e = src.shape[0]
    n_pad = 32 * _TROWS
    mesh = plsc.VectorSubcoreMesh(core_axis_name="c", subcore_axis_name="s")
    fn = pl.kernel(
        functools.partial(_sc_body, n_pad, d, e),
        out_type=[
            jax.ShapeDtypeStruct((n_pad, d), jnp.float32),
            jax.ShapeDtypeStruct((n_pad, d), jnp.float32),
            jax.ShapeDtypeStruct((n_pad,), jnp.float32),
        ],
        mesh=mesh,
        compiler_params=pltpu.CompilerParams(needs_layout_passes=False),
        scratch_types=[
            pltpu.VMEM((_CHUNK,), jnp.int32),
            pltpu.VMEM((_CHUNK,), jnp.int32),
            pltpu.VMEM((_CHUNK,), jnp.int32),
            pltpu.VMEM((_CHUNK,), jnp.int32),
            pltpu.VMEM((_CHUNK + _GB + _VL,), jnp.int32),
            pltpu.VMEM((_CHUNK + _GB + _VL,), jnp.int32),
            pltpu.VMEM((_CAP + 2 * _VL,), jnp.int32),
            pltpu.VMEM((_GB, d), jnp.float32),
            pltpu.VMEM((_TROWS, d), jnp.float32),
            pltpu.VMEM((_TROWS, d), jnp.float32),
            pltpu.VMEM((_TROWS + _VL,), jnp.float32),
            pltpu.VMEM_SHARED((_SROWS, d), jnp.float32),
            pltpu.SemaphoreType.DMA,
            pltpu.SemaphoreType.DMA,
        ],
    )
    return fn(hn, src, dst)


def _final_body(hn_ref, agg_ref, mx_ref, cnt_ref, norm_ref, w_ref, out_ref):
    hn = hn_ref[...]
    agg = agg_ref[...]
    mx = mx_ref[...]
    cnt = cnt_ref[...]
    nr = norm_ref[...]
    w = w_ref[...]

    aggn = agg * nr
    acc1 = jnp.where(cnt > 0.0, mx, 0.0) * nr
    acc3 = (0.2 * agg + 0.5 * cnt) / jnp.maximum(cnt, 1.0) * nr
    x = jnp.concatenate([hn, aggn, acc1, acc3], axis=1)
    y = jnp.dot(x, w, preferred_element_type=jnp.float32)
    out_ref[...] = jnp.maximum(y, 0.0)


def _final(hn, agg, mx, cnt, norm, w, block_rows):
    n, d = hn.shape
    dout = w.shape[1]
    grid = n // block_rows
    return pl.pallas_call(
        _final_body,
        grid=(grid,),
        in_specs=[
            pl.BlockSpec((block_rows, d), lambda i: (i, 0)),
            pl.BlockSpec((block_rows, d), lambda i: (i, 0)),
            pl.BlockSpec((block_rows, d), lambda i: (i, 0)),
            pl.BlockSpec((block_rows, 1), lambda i: (i, 0)),
            pl.BlockSpec((block_rows, 1), lambda i: (i, 0)),
            pl.BlockSpec(w.shape, lambda i: (0, 0)),
        ],
        out_specs=pl.BlockSpec((block_rows, dout), lambda i: (i, 0)),
        out_shape=jax.ShapeDtypeStruct((n, dout), jnp.float32),
    )(hn, agg, mx, cnt, norm, w)


def kernel(h, edge_index, norm, W):
    n, d = h.shape
    src = edge_index[0]
    dst = edge_index[1]
    hn = _prep(h, norm, block_rows=1000)
    hn_pad = jnp.concatenate(
        [hn, jnp.zeros((32 * _TROWS - n, d), jnp.float32)], axis=0)
    agg, mx, cnt = _sc_segment(hn_pad, src, dst)
    out = _final(hn, agg, mx, cnt.reshape(-1, 1), norm, W, block_rows=1000)
    return out


# speedup vs baseline: 1.0567x; 1.0567x over previous
import functools

import jax
import jax.numpy as jnp
from jax import lax
from jax.experimental import pallas as pl
from jax.experimental.pallas import tpu as pltpu
from jax.experimental.pallas import tpu_sc as plsc


def _prep_body(h_ref, norm_ref, hn_ref):
    h = h_ref[...]
    sq = jnp.sum(h * h, axis=-1, keepdims=True)
    hn_ref[...] = h * lax.rsqrt(jnp.maximum(sq, 1e-12)) * norm_ref[...]


def _prep(h, norm, block_rows):
    n, d = h.shape
    grid = n // block_rows
    return pl.pallas_call(
        _prep_body,
        grid=(grid,),
        in_specs=[
            pl.BlockSpec((block_rows, d), lambda i: (i, 0)),
            pl.BlockSpec((block_rows, 1), lambda i: (i, 0)),
        ],
        out_specs=pl.BlockSpec((block_rows, d), lambda i: (i, 0)),
        out_shape=jax.ShapeDtypeStruct((n, d), jnp.float32),
    )(h, norm)


_TROWS = 320
_CHUNK = 1280
_GB = 32
_SROWS = 2560
_CAP = 12288
_SEG = 1600
_VL = 16


def _sc_body(n_pad, d, e, hn, ei, agg_out, mx_out, cnt_out,
             ebufa, ebufb, srcq, rowq, pkg, rows,
             agg_acc, mx_acc, cnt_acc, hn_sh, sem, csem):
    nc = 2
    sid = lax.axis_index("s")
    wid = sid * nc + lax.axis_index("c")
    base = wid * _TROWS
    dsub = d // _VL
    nch = e // _CHUNK
    npair = nch // 2
    nrounds = n_pad // _SROWS

    zf = jnp.zeros((_VL,), jnp.float32)
    ninf = jnp.full((_VL,), -2.0, jnp.float32)

    def init_row(i, _):
        for k in range(dsub):
            agg_acc[i, pl.ds(k * _VL, _VL)] = zf
            mx_acc[i, pl.ds(k * _VL, _VL)] = ninf
        return 0

    lax.fori_loop(0, _TROWS, init_row, 0)

    def init_cnt(i, _):
        cnt_acc[pl.ds(i * _VL, _VL)] = zf
        return 0

    lax.fori_loop(0, (_TROWS + _VL) // _VL, init_cnt, 0)

    one_hot = (lax.iota(jnp.int32, _VL) == 0).astype(jnp.float32)

    def run_batches(qf):
        for t in range(_GB // _VL):
            srcq[pl.ds(qf + t * _VL, _VL)] = jnp.zeros((_VL,), jnp.int32)

        def batch_body(b, _):
            g = b * _GB
            pltpu.async_copy(hn_sh.at[srcq.at[pl.ds(g, _GB)]], rows, sem).wait()
            lim = jnp.minimum(qf - g, _GB)

            def edge_body(j, _):
                rr = rowq[pl.ds(g + j, _VL)][0]
                for k in range(dsub):
                    sl = pl.ds(k * _VL, _VL)
                    m = rows[j, sl]
                    plsc.addupdate(agg_acc.at[rr, sl], m)
                    mx_acc[rr, sl] = jnp.maximum(mx_acc[rr, sl], m)
                plsc.addupdate(cnt_acc.at[pl.ds(rr, _VL)], one_hot)
                return 0

            lax.fori_loop(0, lim, edge_body, 0)
            return 0

        lax.fori_loop(0, (qf + _GB - 1) // _GB, batch_body, 0)

    def stage_block(rbase):
        plsc.subcore_barrier()

        @pl.when(sid == 0)
        def _stage():
            pltpu.sync_copy(hn.at[pl.ds(rbase, _SROWS)], hn_sh)

        plsc.subcore_barrier()

    def filt0_chunk(eb, qn):
        def filt(i, qn):
            for h in range(2):
                sl = pl.ds((2 * i + h) * _VL, _VL)
                s = eb[0, sl]
                delta = eb[1, sl] - base
                mask = plsc.bitcast(delta, jnp.uint32) < jnp.uint32(_TROWS)
                off = jnp.minimum(qn, _CAP)
                pk = s * 512 + delta
                plsc.store_compressed(pkg.at[pl.ds(off, _VL)], pk, mask=mask)
                qn = qn + plsc.all_reduce_population_count(mask)[0]
            return qn

        return lax.fori_loop(0, _CHUNK // (2 * _VL), filt, qn)

    pltpu.async_copy(ei.at[:, pl.ds(0, _CHUNK)], ebufa, csem)

    def pair0_body(i, qn):
        off = i * 2 * _CHUNK
        pltpu.make_async_copy(ei.at[:, pl.ds(off, _CHUNK)], ebufa, csem).wait()
        pltpu.async_copy(ei.at[:, pl.ds(off + _CHUNK, _CHUNK)], ebufb, csem)
        qn = filt0_chunk(ebufa, qn)
        pltpu.make_async_copy(ei.at[:, pl.ds(off + _CHUNK, _CHUNK)], ebufb, csem).wait()

        @pl.when(i + 1 < npair)
        def _prefetch():
            noff = off + 2 * _CHUNK
            pltpu.async_copy(ei.at[:, pl.ds(noff, _CHUNK)], ebufa, csem)

        qn = filt0_chunk(ebufb, qn)
        return qn

    qn_total = lax.fori_loop(0, npair, pair0_body, 0)
    pkg[pl.ds(jnp.minimum(qn_total, _CAP), _VL)] = jnp.full((_VL,), -1, jnp.int32)

    def fast_path():
        def round_body(rnd, _):
            rbase = rnd * _SROWS
            stage_block(rbase)
            nseg = (qn_total + _SEG - 1) // _SEG

            def seg_body(sg, _):
                soff = sg * _SEG
                rem = jnp.minimum(qn_total - soff, _SEG)
                nit = (rem + _VL - 1) // _VL

                def filt(i, qf):
                    pk = pkg[pl.ds(soff + i * _VL, _VL)]
                    s = lax.shift_right_arithmetic(pk, 9) - rbase
                    rw = pk & 511
                    mask = plsc.bitcast(s, jnp.uint32) < jnp.uint32(_SROWS)
                    plsc.store_compressed(srcq.at[pl.ds(qf, _VL)], s, mask=mask)
                    plsc.store_compressed(rowq.at[pl.ds(qf, _VL)], rw, mask=mask)
                    return qf + plsc.all_reduce_population_count(mask)[0]

                qf = lax.fori_loop(0, nit, filt, 0)
                run_batches(qf)
                return 0

            lax.fori_loop(0, nseg, seg_body, 0)
            return 0

        lax.fori_loop(0, nrounds, round_body, 0)

    def slow_path():
        def process_chunk(eb, rbase):
            def filt(i, qf):
                s = eb[0, pl.ds(i * _VL, _VL)] - rbase
                dv = eb[1, pl.ds(i * _VL, _VL)]
                mask = ((dv >= base) & (dv < base + _TROWS)
                        & (s >= 0) & (s < _SROWS))
                plsc.store_compressed(srcq.at[pl.ds(qf, _VL)], s, mask=mask)
                plsc.store_compressed(rowq.at[pl.ds(qf, _VL)], dv - base, mask=mask)
                return qf + plsc.all_reduce_population_count(mask)[0]

            qf = lax.fori_loop(0, _CHUNK // _VL, filt, 0)
            run_batches(qf)

        def round_body(rnd, _):
            rbase = rnd * _SROWS
            stage_block(rbase)
            pltpu.async_copy(ei.at[:, pl.ds(0, _CHUNK)], ebufa, csem)

            def pair_body(i, _):
                off = i * 2 * _CHUNK
                pltpu.make_async_copy(ei.at[:, pl.ds(off, _CHUNK)], ebufa, csem).wait()
                pltpu.async_copy(ei.at[:, pl.ds(off + _CHUNK, _CHUNK)], ebufb, csem)
                process_chunk(ebufa, rbase)
                pltpu.make_async_copy(ei.at[:, pl.ds(off + _CHUNK, _CHUNK)], ebufb, csem).wait()

                @pl.when(i + 1 < npair)
                def _prefetch():
                    noff = off + 2 * _CHUNK
                    pltpu.async_copy(ei.at[:, pl.ds(noff, _CHUNK)], ebufa, csem)

                process_chunk(ebufb, rbase)
                return 0

            lax.fori_loop(0, npair, pair_body, 0)
            return 0

        lax.fori_loop(0, nrounds, round_body, 0)

    lax.cond(qn_total > _CAP, slow_path, fast_path)

    pltpu.sync_copy(agg_acc.at[pl.ds(0, _TROWS)], agg_out.at[pl.ds(base, _TROWS)])
    pltpu.sync_copy(mx_acc.at[pl.ds(0, _TROWS)], mx_out.at[pl.ds(base, _TROWS)])
    pltpu.sync_copy(cnt_acc.at[pl.ds(0, _TROWS)], cnt_out.at[pl.ds(base, _TROWS)])


def _sc_segment(hn, ei):
    n, d = hn.shape
    e = ei.shape[1]
    n_pad = 32 * _TROWS
    mesh = plsc.VectorSubcoreMesh(core_axis_name="c", subcore_axis_name="s")
    fn = pl.kernel(
        functools.partial(_sc_body, n_pad, d, e),
        out_type=[
            jax.ShapeDtypeStruct((n_pad, d), jnp.float32),
            jax.ShapeDtypeStruct((n_pad, d), jnp.float32),
            jax.ShapeDtypeStruct((n_pad,), jnp.float32),
        ],
        mesh=mesh,
        compiler_params=pltpu.CompilerParams(needs_layout_passes=False),
        scratch_types=[
            pltpu.VMEM((2, _CHUNK), jnp.int32),
            pltpu.VMEM((2, _CHUNK), jnp.int32),
            pltpu.VMEM((_SEG + _GB + _VL,), jnp.int32),
            pltpu.VMEM((_SEG + _GB + _VL,), jnp.int32),
            pltpu.VMEM((_CAP + 2 * _VL,), jnp.int32),
            pltpu.VMEM((_GB, d), jnp.float32),
            pltpu.VMEM((_TROWS, d), jnp.float32),
            pltpu.VMEM((_TROWS, d), jnp.float32),
            pltpu.VMEM((_TROWS + _VL,), jnp.float32),
            pltpu.VMEM_SHARED((_SROWS, d), jnp.float32),
            pltpu.SemaphoreType.DMA,
            pltpu.SemaphoreType.DMA,
        ],
    )
    return fn(hn, ei)


def _final_body(hn_ref, agg_ref, mx_ref, cnt_ref, norm_ref, w_ref, out_ref):
    hn = hn_ref[...]
    agg = agg_ref[...]
    mx = mx_ref[...]
    cnt = cnt_ref[...]
    nr = norm_ref[...]
    w = w_ref[...]

    aggn = agg * nr
    acc1 = jnp.where(cnt > 0.0, mx, 0.0) * nr
    acc3 = (0.2 * agg + 0.5 * cnt) / jnp.maximum(cnt, 1.0) * nr
    x = jnp.concatenate([hn, aggn, acc1, acc3], axis=1)
    y = jnp.dot(x, w, preferred_element_type=jnp.float32)
    out_ref[...] = jnp.maximum(y, 0.0)


def _final(hn, agg, mx, cnt, norm, w, block_rows):
    n, d = hn.shape
    dout = w.shape[1]
    grid = n // block_rows
    return pl.pallas_call(
        _final_body,
        grid=(grid,),
        in_specs=[
            pl.BlockSpec((block_rows, d), lambda i: (i, 0)),
            pl.BlockSpec((block_rows, d), lambda i: (i, 0)),
            pl.BlockSpec((block_rows, d), lambda i: (i, 0)),
            pl.BlockSpec((block_rows, 1), lambda i: (i, 0)),
            pl.BlockSpec((block_rows, 1), lambda i: (i, 0)),
            pl.BlockSpec(w.shape, lambda i: (0, 0)),
        ],
        out_specs=pl.BlockSpec((block_rows, dout), lambda i: (i, 0)),
        out_shape=jax.ShapeDtypeStruct((n, dout), jnp.float32),
    )(hn, agg, mx, cnt, norm, w)


def kernel(h, edge_index, norm, W):
    n, d = h.shape
    src = edge_index[0]
    dst = edge_index[1]
    hn = _prep(h, norm, block_rows=1000)
    hn_pad = jnp.concatenate(
        [hn, jnp.zeros((32 * _TROWS - n, d), jnp.float32)], axis=0)
    agg, mx, cnt = _sc_segment(hn_pad, edge_index)
    out = _final(hn, agg, mx, cnt.reshape(-1, 1), norm, W, block_rows=1000)
    return out
